# baseline (speedup 1.0000x reference)
import functools
import numpy as np
import jax
import jax.numpy as jnp
from jax.experimental import pallas as pl

N_MOVIE, N_DIRECTOR, N_ACTOR = 50000, 10000, 30000
HID, HEADS, DH = 128, 8, 16
E = 150000

NB = 2048
EC = 512

NP_M = 51200
NP_D = 10240
NP_A = 30720
EP = 150016


def _proj_kernel(x_ref, w_ref, b_ref, s1_ref, s2_ref, o1_ref, o2_ref):
    h = jnp.dot(x_ref[...], w_ref[...], preferred_element_type=jnp.float32)
    h = h + b_ref[...]
    o1_ref[...] = jnp.dot(h, s1_ref[...], preferred_element_type=jnp.float32)
    o2_ref[...] = jnp.dot(h, s2_ref[...], preferred_element_type=jnp.float32)


def _proj(x, w, b, s1, s2, npad):
    n = x.shape[0]
    xp = jnp.pad(x, ((0, npad - n), (0, 0)))
    grid = (npad // NB,)
    o1, o2 = pl.pallas_call(
        _proj_kernel,
        grid=grid,
        in_specs=[
            pl.BlockSpec((NB, HID), lambda i: (i, 0)),
            pl.BlockSpec((HID, HID), lambda i: (0, 0)),
            pl.BlockSpec((1, HID), lambda i: (0, 0)),
            pl.BlockSpec((HID, HID), lambda i: (0, 0)),
            pl.BlockSpec((HID, HID), lambda i: (0, 0)),
        ],
        out_specs=[
            pl.BlockSpec((NB, HID), lambda i: (i, 0)),
            pl.BlockSpec((NB, HID), lambda i: (i, 0)),
        ],
        out_shape=[
            jax.ShapeDtypeStruct((npad, HID), jnp.float32),
            jax.ShapeDtypeStruct((npad, HID), jnp.float32),
        ],
    )(xp, w, b.reshape(1, HID), s1, s2)
    return o1, o2


def _gather_kernel(rowf_ref, tab_ref, g_ref):
    k = pl.program_id(1)

    @pl.when(k == 0)
    def _():
        g_ref[...] = jnp.zeros_like(g_ref)

    base = k * NB
    ids = base + jax.lax.broadcasted_iota(jnp.int32, (EC, NB), 1)
    onehot = (rowf_ref[...].reshape(EC, 1) == ids).astype(jnp.float32)
    g_ref[...] += jnp.dot(onehot, tab_ref[...],
                          preferred_element_type=jnp.float32)


def _gather(rowf, table):
    npad = table.shape[0]
    w = table.shape[1]
    grid = (EP // EC, npad // NB)
    return pl.pallas_call(
        _gather_kernel,
        grid=grid,
        in_specs=[
            pl.BlockSpec((1, EC), lambda j, k: (0, j)),
            pl.BlockSpec((NB, w), lambda j, k: (k, 0)),
        ],
        out_specs=pl.BlockSpec((EC, w), lambda j, k: (j, 0)),
        out_shape=jax.ShapeDtypeStruct((EP, w), jnp.float32),
    )(rowf, table)


def _scatter_kernel(colf_ref, g_ref, adst_ref, erep_ref, acc_ref, den_ref):
    j = pl.program_id(1)

    @pl.when(j == 0)
    def _():
        acc_ref[...] = jnp.zeros_like(acc_ref)
        den_ref[...] = jnp.zeros_like(den_ref)

    i = pl.program_id(0)
    base = i * NB
    ids = base + jax.lax.broadcasted_iota(jnp.int32, (NB, EC), 0)
    onehot = (ids == colf_ref[...].reshape(1, EC)).astype(jnp.float32)

    g = g_ref[...]
    msg = g[:, :HID]
    asrc = g[:, HID:HID + HEADS]
    adst_blk = adst_ref[:, :HEADS]
    adst_e = jax.lax.dot_general(onehot, adst_blk, (((0,), (0,)), ((), ())),
                                 preferred_element_type=jnp.float32)
    alpha = asrc + adst_e
    alpha = jnp.where(alpha >= 0, alpha, 0.2 * alpha)
    ea = jnp.exp(alpha)
    ea128 = jnp.dot(ea, erep_ref[...], preferred_element_type=jnp.float32)
    acc_ref[...] += jnp.dot(onehot, msg * ea128,
                            preferred_element_type=jnp.float32)
    den_ref[...] += jnp.dot(onehot, ea128,
                            preferred_element_type=jnp.float32)


def _scatter(colf, g, adst, erep):
    grid = (NP_M // NB, EP // EC)
    return pl.pallas_call(
        _scatter_kernel,
        grid=grid,
        in_specs=[
            pl.BlockSpec((1, EC), lambda i, j: (0, j)),
            pl.BlockSpec((EC, 2 * HID), lambda i, j: (j, 0)),
            pl.BlockSpec((NB, HID), lambda i, j: (i, 0)),
            pl.BlockSpec((HEADS, HID), lambda i, j: (0, 0)),
        ],
        out_specs=[
            pl.BlockSpec((NB, HID), lambda i, j: (i, 0)),
            pl.BlockSpec((NB, HID), lambda i, j: (i, 0)),
        ],
        out_shape=[
            jax.ShapeDtypeStruct((NP_M, HID), jnp.float32),
            jax.ShapeDtypeStruct((NP_M, HID), jnp.float32),
        ],
    )(colf, g, adst, erep)


def _group_kernel(a1_ref, d1_ref, a2_ref, d2_ref, kw_ref, kb_ref,
                  o1_ref, o2_ref, ks_ref):
    i = pl.program_id(0)

    @pl.when(i == 0)
    def _():
        ks_ref[...] = jnp.zeros_like(ks_ref)

    o1 = jnp.maximum(a1_ref[...] / (d1_ref[...] + 1e-16), 0.0)
    o2 = jnp.maximum(a2_ref[...] / (d2_ref[...] + 1e-16), 0.0)
    o1_ref[...] = o1
    o2_ref[...] = o2
    t1 = jnp.sum(jnp.tanh(jnp.dot(o1, kw_ref[...],
                                  preferred_element_type=jnp.float32)
                          + kb_ref[...]), axis=0, keepdims=True)
    t2 = jnp.sum(jnp.tanh(jnp.dot(o2, kw_ref[...],
                                  preferred_element_type=jnp.float32)
                          + kb_ref[...]), axis=0, keepdims=True)
    rows = jax.lax.broadcasted_iota(jnp.int32, (8, 1), 0)
    e0 = (rows == 0).astype(jnp.float32)
    e1 = (rows == 1).astype(jnp.float32)
    ks_ref[...] += (jnp.dot(e0, t1, preferred_element_type=jnp.float32)
                    + jnp.dot(e1, t2, preferred_element_type=jnp.float32))


def _group(acc1, den1, acc2, den2, kw, kb):
    grid = (NP_M // NB,)
    return pl.pallas_call(
        _group_kernel,
        grid=grid,
        in_specs=[
            pl.BlockSpec((NB, HID), lambda i: (i, 0)),
            pl.BlockSpec((NB, HID), lambda i: (i, 0)),
            pl.BlockSpec((NB, HID), lambda i: (i, 0)),
            pl.BlockSpec((NB, HID), lambda i: (i, 0)),
            pl.BlockSpec((HID, HID), lambda i: (0, 0)),
            pl.BlockSpec((1, HID), lambda i: (0, 0)),
        ],
        out_specs=[
            pl.BlockSpec((NB, HID), lambda i: (i, 0)),
            pl.BlockSpec((NB, HID), lambda i: (i, 0)),
            pl.BlockSpec((8, HID), lambda i: (0, 0)),
        ],
        out_shape=[
            jax.ShapeDtypeStruct((NP_M, HID), jnp.float32),
            jax.ShapeDtypeStruct((NP_M, HID), jnp.float32),
            jax.ShapeDtypeStruct((8, HID), jnp.float32),
        ],
    )(acc1, den1, acc2, den2, kw, kb.reshape(1, HID))


def _final_kernel(o1_ref, o2_ref, a0_ref, a1_ref, lw_ref, lb_ref, out_ref):
    gm = o1_ref[...] * a0_ref[...] + o2_ref[...] * a1_ref[...]
    out_ref[...] = jnp.dot(gm, lw_ref[...],
                           preferred_element_type=jnp.float32) + lb_ref[...]


def _final(o1, o2, a0, a1, lw, lb):
    grid = (NP_M // NB,)
    return pl.pallas_call(
        _final_kernel,
        grid=grid,
        in_specs=[
            pl.BlockSpec((NB, HID), lambda i: (i, 0)),
            pl.BlockSpec((NB, HID), lambda i: (i, 0)),
            pl.BlockSpec((1, HID), lambda i: (0, 0)),
            pl.BlockSpec((1, HID), lambda i: (0, 0)),
            pl.BlockSpec((HID, HID), lambda i: (0, 0)),
            pl.BlockSpec((1, HID), lambda i: (0, 0)),
        ],
        out_specs=pl.BlockSpec((NB, HID), lambda i: (i, 0)),
        out_shape=jax.ShapeDtypeStruct((NP_M, HID), jnp.float32),
    )(o1, o2, a0, a1, lw, lb)


def _att_mat(att):
    a = att.reshape(HEADS, DH)
    blocks = [a[h].reshape(DH, 1) for h in range(HEADS)]
    m = jax.scipy.linalg.block_diag(*blocks)
    return jnp.pad(m, ((0, 0), (0, HID - HEADS)))


_EREP = jnp.asarray(np.kron(np.eye(HEADS, dtype=np.float32),
                            np.ones((1, DH), dtype=np.float32)))
_EYE = jnp.eye(HID, dtype=jnp.float32)


def _pad_edges(idx):
    pad = jnp.full((EP - E,), 10_000_000, dtype=idx.dtype)
    return jnp.concatenate([idx, pad]).astype(jnp.int32).reshape(1, EP)


@jax.jit
def kernel(x_movie, x_director, x_actor, ei_md, ei_dm, ei_ma, ei_am,
           proj_w_movie, proj_b_movie, proj_w_director, proj_b_director,
           proj_w_actor, proj_b_actor,
           att_src_md, att_dst_md, att_src_dm, att_dst_dm,
           att_src_ma, att_dst_ma, att_src_am, att_dst_am,
           k_w, k_b, q, lin_w, lin_b):
    sc_dm, sc_am = _proj(x_movie, proj_w_movie, proj_b_movie,
                         _att_mat(att_dst_dm), _att_mat(att_dst_am), NP_M)
    h_d, as_d = _proj(x_director, proj_w_director, proj_b_director,
                      _EYE, _att_mat(att_src_dm), NP_D)
    h_a, as_a = _proj(x_actor, proj_w_actor, proj_b_actor,
                      _EYE, _att_mat(att_src_am), NP_A)

    tab_d = jnp.concatenate([h_d, as_d], axis=1)
    tab_a = jnp.concatenate([h_a, as_a], axis=1)

    row_dm = _pad_edges(ei_dm[0])
    col_dm = _pad_edges(ei_dm[1])
    row_am = _pad_edges(ei_am[0])
    col_am = _pad_edges(ei_am[1])

    g_dm = _gather(row_dm, tab_d)
    g_am = _gather(row_am, tab_a)

    acc1, den1 = _scatter(col_dm, g_dm, sc_dm, _EREP)
    acc2, den2 = _scatter(col_am, g_am, sc_am, _EREP)

    o1, o2, ksum = _group(acc1, den1, acc2, den2, k_w, k_b)

    ksc = ksum[:2] / N_MOVIE
    logits = (q[None, :] * ksc).sum(-1)
    attn = jax.nn.softmax(logits)
    a0 = jnp.full((1, HID), attn[0], dtype=jnp.float32)
    a1 = jnp.full((1, HID), attn[1], dtype=jnp.float32)

    lw = jnp.pad(lin_w, ((0, 0), (0, HID - lin_w.shape[1])))
    lb = jnp.pad(lin_b, (0, HID - lin_b.shape[0])).reshape(1, HID)

    out = _final(o1, o2, a0, a1, lw, lb)
    return out[:N_MOVIE, :lin_w.shape[1]]
